# baseline (speedup 1.0000x reference)
import functools

import numpy as np

import jax
import jax.numpy as jnp
from jax import lax
from jax.experimental import pallas as pl
from jax.experimental.pallas import tpu as pltpu
from jax.experimental.pallas import tpu_sc as plsc

K = 16
N_BLOCKS = 7

L = 16
NW = 32
CAPV = 16
CAP = CAPV * L
GV = 4
BLK = GV * L
TRIG = CAP - BLK
BUF = CAP + L

_INF = np.float32(np.inf)


def _vsort(k, v):
    return plsc.sort_key_val(k, v)


def _cmp_swap(ka, va, kb, vb):
    m = ka <= kb
    return (jnp.where(m, ka, kb), jnp.where(m, va, vb),
            jnp.where(m, kb, ka), jnp.where(m, vb, va))


def _bitonic_merge(ks, vs):
    m = len(ks)
    if m == 1:
        k2, v2 = _vsort(ks[0], vs[0])
        return [k2], [v2]
    h = m // 2
    lo_k, lo_v, hi_k, hi_v = [], [], [], []
    for i in range(h):
        lk, lv, hk, hv = _cmp_swap(ks[i], vs[i], ks[i + h], vs[i + h])
        lo_k.append(lk); lo_v.append(lv); hi_k.append(hk); hi_v.append(hv)
    ak, av = _bitonic_merge(lo_k, lo_v)
    bk, bv = _bitonic_merge(hi_k, hi_v)
    return ak + bk, av + bv


def _merge_sorted(aks, avs, bks, bvs, cap):
    rk = [lax.rev(k, (0,)) for k in reversed(bks)]
    rv = [lax.rev(v, (0,)) for v in reversed(bvs)]
    ks = aks + rk
    vs = avs + rv
    while len(ks) // 2 >= cap and len(ks) > 1:
        h = len(ks) // 2
        nk, nv = [], []
        for i in range(h):
            lk, lv, _, _ = _cmp_swap(ks[i], vs[i], ks[i + h], vs[i + h])
            nk.append(lk); nv.append(lv)
        ks, vs = nk, nv
    return _bitonic_merge(ks, vs)


def _select_top(buf_k, buf_i, ptr, out_vregs):
    ks, vs = [], []
    for g in range(CAPV):
        k = buf_k[pl.ds(g * L, L)]
        v = buf_i[pl.ds(g * L, L)]
        pos = lax.iota(jnp.int32, L) + g * L
        k = jnp.where(pos < ptr, k, _INF)
        sk, sv = _vsort(k, v)
        ks.append([sk]); vs.append([sv])
    while len(ks) > 1:
        nk, nv = [], []
        for i in range(0, len(ks), 2):
            a, b = _merge_sorted(ks[i], vs[i], ks[i + 1], vs[i + 1],
                                 cap=max(out_vregs, 1))
            nk.append(a); nv.append(b)
        ks, vs = nk, nv
    return ks[0][:out_vregs], vs[0][:out_vregs]


def _lane_extract_f32(v, lane):
    sel = lax.iota(jnp.int32, L) == lane
    return jnp.max(jnp.where(sel, v, -_INF))


SLACK = 2
FINV = 8
TIE_PASSES = 6
IMAX = np.int32(2**31 - 1)


@functools.cache
def _make_knn_select(BN, N, d):
    T = 16 * d
    TV = T // L
    KEEP = TV + SLACK
    rows_per_w = BN // NW
    steps = N // L
    mesh = plsc.VectorSubcoreMesh(core_axis_name="c", subcore_axis_name="s")

    @functools.partial(
        pl.kernel,
        out_type=jax.ShapeDtypeStruct((BN, L), jnp.int32),
        mesh=mesh,
        compiler_params=pltpu.CompilerParams(needs_layout_passes=False),
        scratch_types=[
            pltpu.VMEM((N,), jnp.float32),
            pltpu.VMEM((BUF,), jnp.float32),
            pltpu.VMEM((BUF,), jnp.int32),
            pltpu.VMEM((144,), jnp.float32),
            pltpu.VMEM((144,), jnp.int32),
            pltpu.VMEM((L,), jnp.int32),
        ],
    )
    def knn(dist_hbm, out_hbm, row_v, bk, bi, stage_k, stage_i, out_v):
        wid = lax.axis_index("s") * 2 + lax.axis_index("c")
        row0 = wid * rows_per_w

        def do_row(r, carry):
            row = row0 + r
            pltpu.sync_copy(dist_hbm.at[row], row_v)

            def mstep(j, accs):
                return tuple(
                    jnp.maximum(a, row_v[pl.ds((j * 16 + g) * L, L)])
                    for g, a in enumerate(accs))

            accs = lax.fori_loop(
                1, 16, mstep,
                tuple(row_v[pl.ds(g * L, L)] for g in range(16)))
            runs = [jnp.sort(a) for a in accs]
            while len(runs) > 1:
                nr = []
                for i in range(0, len(runs), 2):
                    nr.append(jnp.sort(jnp.minimum(
                        runs[i], lax.rev(runs[i + 1], (0,)))))
                runs = nr
            thr0 = _lane_extract_f32(runs[0], d - 1)

            def reselect(ptr, thr):
                ks, vs = _select_top(bk, bi, ptr, KEEP)
                for g in range(KEEP):
                    bk[pl.ds(g * L, L)] = ks[g]
                    bi[pl.ds(g * L, L)] = vs[g]
                new_thr = _lane_extract_f32(ks[(T - 1) // L], (T - 1) % L)
                return np.int32(KEEP * L), new_thr

            def step(s, sc):
                ptr, thr = sc
                base = s * BLK
                vals = [row_v[pl.ds(base + g * L, L)] for g in range(GV)]
                masks = [v <= thr for v in vals]
                anym = masks[0]
                for m in masks[1:]:
                    anym = anym | m

                def append(p, t):
                    off = p + jnp.zeros((L,), jnp.int32)
                    trash = CAP + lax.iota(jnp.int32, L)
                    for g in range(GV):
                        m = masks[g]
                        c = plsc.cumsum(m.astype(jnp.int32))
                        cnt = plsc.all_reduce_population_count(m)
                        idx = lax.iota(jnp.int32, L) + (base + g * L)
                        dest = jnp.where(m, off + c - 1, trash)
                        plsc.store_scatter(bk, [dest], vals[g], mask=m)
                        plsc.store_scatter(bi, [dest], idx, mask=m)
                        off = off + cnt
                    p2 = jnp.max(off)
                    return lax.cond(p2 >= TRIG, reselect,
                                    lambda a, b: (a, b), p2, t)

                return lax.cond(jnp.any(anym), append,
                                lambda a, b: (a, b), ptr, thr)

            ptr, _thr = lax.fori_loop(0, N // BLK, step, (np.int32(0), thr0))

            ks, vs = _select_top(bk, bi, ptr, FINV)
            stage_k[pl.ds(8 * L, L)] = jnp.full((L,), _INF, jnp.float32)
            stage_i[pl.ds(8 * L, L)] = jnp.full((L,), IMAX, jnp.int32)
            stage_k[pl.ds(0, L)] = jnp.full((L,), -_INF, jnp.float32)
            stage_i[pl.ds(0, L)] = jnp.full((L,), IMAX, jnp.int32)
            for g in range(FINV):
                stage_k[pl.ds(g * L + 1, L)] = ks[g]
                stage_i[pl.ds(g * L + 1, L)] = vs[g]
            par0 = lax.iota(jnp.int32, L) % 2
            for p in range(TIE_PASSES):
                par = (par0 + p) % 2 == 0
                new_is = []
                for g in range(FINV):
                    pk = stage_k[pl.ds(g * L, L)]
                    ck = stage_k[pl.ds(g * L + 1, L)]
                    nk = stage_k[pl.ds(g * L + 2, L)]
                    pi = stage_i[pl.ds(g * L, L)]
                    ci = stage_i[pl.ds(g * L + 1, L)]
                    ni = stage_i[pl.ds(g * L + 2, L)]
                    take_next = par & (ck == nk) & (ci > ni)
                    take_prev = (~par) & (pk == ck) & (pi > ci)
                    new_is.append(jnp.where(
                        take_prev, pi, jnp.where(take_next, ni, ci)))
                for g in range(FINV):
                    stage_i[pl.ds(g * L + 1, L)] = new_is[g]
            picks = lax.iota(jnp.int32, L) * d + 1
            out_v[...] = plsc.load_gather(stage_i, [picks])
            pltpu.sync_copy(out_v, out_hbm.at[row])
            return carry

        lax.fori_loop(0, rows_per_w, do_row, 0)

    return knn


def _knn(x, d):
    B, C, N, _ = x.shape
    xt = jnp.transpose(x[:, :, :, 0], (0, 2, 1))
    x2 = jnp.sum(xt * xt, axis=-1, keepdims=True)
    dist = x2 - 2.0 * jnp.einsum('bnc,bmc->bnm', xt, xt) + jnp.transpose(x2, (0, 2, 1))
    nn = _make_knn_select(B * N, N, d)(dist.reshape(B * N, N))
    return nn.reshape(B, N, K)


@functools.cache
def _make_gather(B, C, N):
    PAIRS = B * C
    PPW = -(-PAIRS // NW)
    NK = N * K
    CHUNK = 4096
    mesh = plsc.VectorSubcoreMesh(core_axis_name="c", subcore_axis_name="s")

    @functools.partial(
        pl.kernel,
        out_type=jax.ShapeDtypeStruct((PAIRS, NK), jnp.float32),
        mesh=mesh,
        compiler_params=pltpu.CompilerParams(needs_layout_passes=False),
        scratch_types=[
            pltpu.VMEM((NK,), jnp.int32),
            pltpu.VMEM((N,), jnp.float32),
            pltpu.VMEM((CHUNK,), jnp.float32),
        ],
    )
    def gth(x_hbm, idx_hbm, out_hbm, idx_v, tab_v, stage_v):
        wid = lax.axis_index("s") * 2 + lax.axis_index("c")

        def do_pair(pi, carry):
            pair = wid * PPW + pi

            def work(_):
                b = pair // C
                pltpu.sync_copy(idx_hbm.at[b], idx_v)
                pltpu.sync_copy(x_hbm.at[pair], tab_v)

                def do_chunk(ci, c2):
                    def do_grp(t, c3):
                        base = t * 4 * L
                        for u in range(4):
                            iv = idx_v[pl.ds(ci * CHUNK + base + u * L, L)]
                            stage_v[pl.ds(base + u * L, L)] = (
                                plsc.load_gather(tab_v, [iv]))
                        return c3

                    lax.fori_loop(0, CHUNK // (4 * L), do_grp, 0)
                    pltpu.sync_copy(stage_v,
                                    out_hbm.at[pair, pl.ds(ci * CHUNK, CHUNK)])
                    return c2

                lax.fori_loop(0, NK // CHUNK, do_chunk, 0)
                return 0

            if PAIRS % NW:
                lax.cond(pair < PAIRS, work, lambda _: 0, 0)
            else:
                work(0)
            return carry

        lax.fori_loop(0, PPW, do_pair, 0)

    return gth


def _gather(x, idx):
    B, C, N, _ = x.shape
    out = _make_gather(B, C, N)(x[:, :, :, 0].reshape(B * C, N),
                                idx.reshape(B, N * K))
    return out.reshape(B, C, N, K)


def _conv(x, W, b):
    return jnp.einsum('bcnk,oc->bonk', x, W) + b[None, :, None, None]


def _bn(x):
    m = jnp.mean(x, axis=(0, 2, 3), keepdims=True)
    v = jnp.mean((x - m) ** 2, axis=(0, 2, 3), keepdims=True)
    return (x - m) / jnp.sqrt(v + 1e-5)


def _edge_conv(x, nn_idx, W, b):
    xj = _gather(x, nn_idx)
    xi = jnp.broadcast_to(x, xj.shape)
    h = jnp.concatenate([xi, xj - xi], axis=1)
    h = jax.nn.relu(_bn(_conv(h, W, b)))
    return jnp.max(h, axis=-1, keepdims=True)


def _final_conv_body(x_ref, w_ref, b_ref, o_ref):
    o_ref[...] = jnp.dot(x_ref[...], w_ref[...],
                         preferred_element_type=jnp.float32) + b_ref[...]


def _final_conv(h, W, b):
    B, C, N, _ = h.shape
    O = W.shape[0]
    x = jnp.transpose(h[:, :, :, 0], (0, 2, 1)).reshape(B * N, C)
    out = pl.pallas_call(
        _final_conv_body,
        out_shape=jax.ShapeDtypeStruct((B * N, O), jnp.float32),
        grid=(B * N // 2048,),
        in_specs=[
            pl.BlockSpec((2048, C), lambda i: (i, 0)),
            pl.BlockSpec((C, O), lambda i: (0, 0)),
            pl.BlockSpec((1, O), lambda i: (0, 0)),
        ],
        out_specs=pl.BlockSpec((2048, O), lambda i: (i, 0)),
    )(x, W.T, b.reshape(1, O))
    return out.reshape(B, N, O)


def kernel(inputs, W_head, b_head, W_blk, b_blk, W_fus, b_fus, W_p1, b_p1, W_p2, b_p2, W_p3, b_p3):
    nn_idx = _knn(inputs[:, 0:3], 1)
    x = _edge_conv(inputs, nn_idx, W_head, b_head)
    feats = [x]
    for i in range(N_BLOCKS - 1):
        xin = feats[-1]
        idx = _knn(xin, 1 + i)
        feats.append(_edge_conv(xin, idx, W_blk[i], b_blk[i]) + xin)
    feats = jnp.concatenate(feats, axis=1)
    fusion = jax.nn.relu(_bn(_conv(feats, W_fus, b_fus)))
    fusion = jnp.max(fusion, axis=(2, 3), keepdims=True)
    fusion = jnp.broadcast_to(fusion, (fusion.shape[0], fusion.shape[1], feats.shape[2], 1))
    h = jnp.concatenate([fusion, feats], axis=1)
    h = jax.nn.relu(_bn(_conv(h, W_p1, b_p1)))
    h = jax.nn.relu(_bn(_conv(h, W_p2, b_p2)))
    return _final_conv(h, W_p3, b_p3)


# speedup vs baseline: 1.0358x; 1.0358x over previous
import functools

import numpy as np

import jax
import jax.numpy as jnp
from jax import lax
from jax.experimental import pallas as pl
from jax.experimental.pallas import tpu as pltpu
from jax.experimental.pallas import tpu_sc as plsc

K = 16
N_BLOCKS = 7

L = 16
NW = 32
CAPV = 16
CAP = CAPV * L
GV = 4
BLK = GV * L
TRIG = CAP - BLK
BUF = CAP + L

_INF = np.float32(np.inf)


def _vsort(k, v):
    return plsc.sort_key_val(k, v)


def _cmp_swap(ka, va, kb, vb):
    m = ka <= kb
    return (jnp.where(m, ka, kb), jnp.where(m, va, vb),
            jnp.where(m, kb, ka), jnp.where(m, vb, va))


def _bitonic_merge(ks, vs):
    m = len(ks)
    if m == 1:
        k2, v2 = _vsort(ks[0], vs[0])
        return [k2], [v2]
    h = m // 2
    lo_k, lo_v, hi_k, hi_v = [], [], [], []
    for i in range(h):
        lk, lv, hk, hv = _cmp_swap(ks[i], vs[i], ks[i + h], vs[i + h])
        lo_k.append(lk); lo_v.append(lv); hi_k.append(hk); hi_v.append(hv)
    ak, av = _bitonic_merge(lo_k, lo_v)
    bk, bv = _bitonic_merge(hi_k, hi_v)
    return ak + bk, av + bv


def _merge_sorted(aks, avs, bks, bvs, cap):
    rk = [lax.rev(k, (0,)) for k in reversed(bks)]
    rv = [lax.rev(v, (0,)) for v in reversed(bvs)]
    ks = aks + rk
    vs = avs + rv
    while len(ks) // 2 >= cap and len(ks) > 1:
        h = len(ks) // 2
        nk, nv = [], []
        for i in range(h):
            lk, lv, _, _ = _cmp_swap(ks[i], vs[i], ks[i + h], vs[i + h])
            nk.append(lk); nv.append(lv)
        ks, vs = nk, nv
    return _bitonic_merge(ks, vs)


def _select_top(buf_k, buf_i, ptr, out_vregs):
    ks, vs = [], []
    for g in range(CAPV):
        k = buf_k[pl.ds(g * L, L)]
        v = buf_i[pl.ds(g * L, L)]
        pos = lax.iota(jnp.int32, L) + g * L
        k = jnp.where(pos < ptr, k, _INF)
        sk, sv = _vsort(k, v)
        ks.append([sk]); vs.append([sv])
    while len(ks) > 1:
        nk, nv = [], []
        for i in range(0, len(ks), 2):
            a, b = _merge_sorted(ks[i], vs[i], ks[i + 1], vs[i + 1],
                                 cap=max(out_vregs, 1))
            nk.append(a); nv.append(b)
        ks, vs = nk, nv
    return ks[0][:out_vregs], vs[0][:out_vregs]


def _lane_extract_f32(v, lane):
    sel = lax.iota(jnp.int32, L) == lane
    return jnp.max(jnp.where(sel, v, -_INF))


SLACK = 2
FINV = 8
TIE_PASSES = 6
IMAX = np.int32(2**31 - 1)


@functools.cache
def _make_knn_select(BN, N, d):
    T = 16 * d
    TV = T // L
    KEEP = TV + SLACK
    rows_per_w = BN // NW
    steps = N // L
    mesh = plsc.VectorSubcoreMesh(core_axis_name="c", subcore_axis_name="s")

    @functools.partial(
        pl.kernel,
        out_type=jax.ShapeDtypeStruct((BN, L), jnp.int32),
        mesh=mesh,
        compiler_params=pltpu.CompilerParams(needs_layout_passes=False),
        scratch_types=[
            pltpu.VMEM((N,), jnp.float32),
            pltpu.VMEM((BUF,), jnp.float32),
            pltpu.VMEM((BUF,), jnp.int32),
            pltpu.VMEM((144,), jnp.float32),
            pltpu.VMEM((144,), jnp.int32),
            pltpu.VMEM((L,), jnp.int32),
        ],
    )
    def knn(dist_hbm, out_hbm, row_v, bk, bi, stage_k, stage_i, out_v):
        wid = lax.axis_index("s") * 2 + lax.axis_index("c")
        row0 = wid * rows_per_w

        def do_row(r, carry):
            row = row0 + r
            pltpu.sync_copy(dist_hbm.at[row], row_v)

            def reselect(ptr, thr):
                ks, vs = _select_top(bk, bi, ptr, KEEP)
                for g in range(KEEP):
                    bk[pl.ds(g * L, L)] = ks[g]
                    bi[pl.ds(g * L, L)] = vs[g]
                new_thr = _lane_extract_f32(ks[(T - 1) // L], (T - 1) % L)
                return np.int32(KEEP * L), new_thr

            def step(s, sc):
                ptr, thr = sc
                base = s * BLK
                vals = [row_v[pl.ds(base + g * L, L)] for g in range(GV)]
                masks = [v <= thr for v in vals]
                anym = masks[0]
                for m in masks[1:]:
                    anym = anym | m

                def append(p, t):
                    off = p + jnp.zeros((L,), jnp.int32)
                    trash = CAP + lax.iota(jnp.int32, L)
                    for g in range(GV):
                        m = masks[g]
                        c = plsc.cumsum(m.astype(jnp.int32))
                        cnt = plsc.all_reduce_population_count(m)
                        idx = lax.iota(jnp.int32, L) + (base + g * L)
                        dest = jnp.where(m, off + c - 1, trash)
                        plsc.store_scatter(bk, [dest], vals[g], mask=m)
                        plsc.store_scatter(bi, [dest], idx, mask=m)
                        off = off + cnt
                    p2 = jnp.max(off)
                    return lax.cond(p2 >= TRIG, reselect,
                                    lambda a, b: (a, b), p2, t)

                return lax.cond(jnp.any(anym), append,
                                lambda a, b: (a, b), ptr, thr)

            ptr, _thr = lax.fori_loop(0, N // BLK, step, (np.int32(0), _INF))

            ks, vs = _select_top(bk, bi, ptr, KEEP)
            stage_k[pl.ds(KEEP * L, L)] = jnp.full((L,), _INF, jnp.float32)
            stage_i[pl.ds(KEEP * L, L)] = jnp.full((L,), IMAX, jnp.int32)
            stage_k[pl.ds(0, L)] = jnp.full((L,), -_INF, jnp.float32)
            stage_i[pl.ds(0, L)] = jnp.full((L,), IMAX, jnp.int32)
            for g in range(KEEP):
                stage_k[pl.ds(g * L + 1, L)] = ks[g]
                stage_i[pl.ds(g * L + 1, L)] = vs[g]
            par0 = lax.iota(jnp.int32, L) % 2
            for p in range(TIE_PASSES):
                par = (par0 + p) % 2 == 0
                new_is = []
                for g in range(KEEP):
                    pk = stage_k[pl.ds(g * L, L)]
                    ck = stage_k[pl.ds(g * L + 1, L)]
                    nk = stage_k[pl.ds(g * L + 2, L)]
                    pi = stage_i[pl.ds(g * L, L)]
                    ci = stage_i[pl.ds(g * L + 1, L)]
                    ni = stage_i[pl.ds(g * L + 2, L)]
                    take_next = par & (ck == nk) & (ci > ni)
                    take_prev = (~par) & (pk == ck) & (pi > ci)
                    new_is.append(jnp.where(
                        take_prev, pi, jnp.where(take_next, ni, ci)))
                for g in range(KEEP):
                    stage_i[pl.ds(g * L + 1, L)] = new_is[g]
            picks = lax.iota(jnp.int32, L) * d + 1
            out_v[...] = plsc.load_gather(stage_i, [picks])
            pltpu.sync_copy(out_v, out_hbm.at[row])
            return carry

        lax.fori_loop(0, rows_per_w, do_row, 0)

    return knn


def _knn(x, d):
    B, C, N, _ = x.shape
    xt = jnp.transpose(x[:, :, :, 0], (0, 2, 1))
    x2 = jnp.sum(xt * xt, axis=-1, keepdims=True)
    dist = x2 - 2.0 * jnp.einsum('bnc,bmc->bnm', xt, xt) + jnp.transpose(x2, (0, 2, 1))
    nn = _make_knn_select(B * N, N, d)(dist.reshape(B * N, N))
    return nn.reshape(B, N, K)


@functools.cache
def _make_gather(B, C, N):
    PAIRS = B * C
    PPW = -(-PAIRS // NW)
    NK = N * K
    CHUNK = 4096
    mesh = plsc.VectorSubcoreMesh(core_axis_name="c", subcore_axis_name="s")

    @functools.partial(
        pl.kernel,
        out_type=jax.ShapeDtypeStruct((PAIRS, NK), jnp.float32),
        mesh=mesh,
        compiler_params=pltpu.CompilerParams(needs_layout_passes=False),
        scratch_types=[
            pltpu.VMEM((NK,), jnp.int32),
            pltpu.VMEM((N,), jnp.float32),
            pltpu.VMEM((CHUNK,), jnp.float32),
        ],
    )
    def gth(x_hbm, idx_hbm, out_hbm, idx_v, tab_v, stage_v):
        wid = lax.axis_index("s") * 2 + lax.axis_index("c")

        def do_pair(pi, carry):
            pair = wid * PPW + pi

            def work(_):
                b = pair // C
                pltpu.sync_copy(idx_hbm.at[b], idx_v)
                pltpu.sync_copy(x_hbm.at[pair], tab_v)

                def do_chunk(ci, c2):
                    def do_grp(t, c3):
                        base = t * 4 * L
                        for u in range(4):
                            iv = idx_v[pl.ds(ci * CHUNK + base + u * L, L)]
                            stage_v[pl.ds(base + u * L, L)] = (
                                plsc.load_gather(tab_v, [iv]))
                        return c3

                    lax.fori_loop(0, CHUNK // (4 * L), do_grp, 0)
                    pltpu.sync_copy(stage_v,
                                    out_hbm.at[pair, pl.ds(ci * CHUNK, CHUNK)])
                    return c2

                lax.fori_loop(0, NK // CHUNK, do_chunk, 0)
                return 0

            if PAIRS % NW:
                lax.cond(pair < PAIRS, work, lambda _: 0, 0)
            else:
                work(0)
            return carry

        lax.fori_loop(0, PPW, do_pair, 0)

    return gth


def _gather(x, idx):
    B, C, N, _ = x.shape
    out = _make_gather(B, C, N)(x[:, :, :, 0].reshape(B * C, N),
                                idx.reshape(B, N * K))
    return out.reshape(B, C, N, K)


def _conv(x, W, b):
    return jnp.einsum('bcnk,oc->bonk', x, W) + b[None, :, None, None]


def _bn(x):
    m = jnp.mean(x, axis=(0, 2, 3), keepdims=True)
    v = jnp.mean((x - m) ** 2, axis=(0, 2, 3), keepdims=True)
    return (x - m) / jnp.sqrt(v + 1e-5)


def _edge_conv(x, nn_idx, W, b):
    xj = _gather(x, nn_idx)
    xi = jnp.broadcast_to(x, xj.shape)
    h = jnp.concatenate([xi, xj - xi], axis=1)
    h = jax.nn.relu(_bn(_conv(h, W, b)))
    return jnp.max(h, axis=-1, keepdims=True)


def _final_conv_body(x_ref, w_ref, b_ref, o_ref):
    o_ref[...] = jnp.dot(x_ref[...], w_ref[...],
                         preferred_element_type=jnp.float32) + b_ref[...]


def _final_conv(h, W, b):
    B, C, N, _ = h.shape
    O = W.shape[0]
    x = jnp.transpose(h[:, :, :, 0], (0, 2, 1)).reshape(B * N, C)
    out = pl.pallas_call(
        _final_conv_body,
        out_shape=jax.ShapeDtypeStruct((B * N, O), jnp.float32),
        grid=(B * N // 2048,),
        in_specs=[
            pl.BlockSpec((2048, C), lambda i: (i, 0)),
            pl.BlockSpec((C, O), lambda i: (0, 0)),
            pl.BlockSpec((1, O), lambda i: (0, 0)),
        ],
        out_specs=pl.BlockSpec((2048, O), lambda i: (i, 0)),
    )(x, W.T, b.reshape(1, O))
    return out.reshape(B, N, O)


def kernel(inputs, W_head, b_head, W_blk, b_blk, W_fus, b_fus, W_p1, b_p1, W_p2, b_p2, W_p3, b_p3):
    nn_idx = _knn(inputs[:, 0:3], 1)
    x = _edge_conv(inputs, nn_idx, W_head, b_head)
    feats = [x]
    for i in range(N_BLOCKS - 1):
        xin = feats[-1]
        idx = _knn(xin, 1 + i)
        feats.append(_edge_conv(xin, idx, W_blk[i], b_blk[i]) + xin)
    feats = jnp.concatenate(feats, axis=1)
    fusion = jax.nn.relu(_bn(_conv(feats, W_fus, b_fus)))
    fusion = jnp.max(fusion, axis=(2, 3), keepdims=True)
    fusion = jnp.broadcast_to(fusion, (fusion.shape[0], fusion.shape[1], feats.shape[2], 1))
    h = jnp.concatenate([fusion, feats], axis=1)
    h = jax.nn.relu(_bn(_conv(h, W_p1, b_p1)))
    h = jax.nn.relu(_bn(_conv(h, W_p2, b_p2)))
    return _final_conv(h, W_p3, b_p3)


# speedup vs baseline: 1.1703x; 1.1299x over previous
import functools

import numpy as np

import jax
import jax.numpy as jnp
from jax import lax
from jax.experimental import pallas as pl
from jax.experimental.pallas import tpu as pltpu
from jax.experimental.pallas import tpu_sc as plsc

K = 16
N_BLOCKS = 7

L = 16
NW = 32
CAPV = 16
CAP = CAPV * L
GV = 4
BLK = GV * L
TRIG = CAP - BLK
BUF = CAP + L

_INF = np.float32(np.inf)


def _vsort(k, v):
    return plsc.sort_key_val(k, v)


def _cmp_swap(ka, va, kb, vb):
    m = ka <= kb
    return (jnp.where(m, ka, kb), jnp.where(m, va, vb),
            jnp.where(m, kb, ka), jnp.where(m, vb, va))


def _bitonic_merge(ks, vs):
    m = len(ks)
    if m == 1:
        k2, v2 = _vsort(ks[0], vs[0])
        return [k2], [v2]
    h = m // 2
    lo_k, lo_v, hi_k, hi_v = [], [], [], []
    for i in range(h):
        lk, lv, hk, hv = _cmp_swap(ks[i], vs[i], ks[i + h], vs[i + h])
        lo_k.append(lk); lo_v.append(lv); hi_k.append(hk); hi_v.append(hv)
    ak, av = _bitonic_merge(lo_k, lo_v)
    bk, bv = _bitonic_merge(hi_k, hi_v)
    return ak + bk, av + bv


def _merge_sorted(aks, avs, bks, bvs, cap):
    rk = [lax.rev(k, (0,)) for k in reversed(bks)]
    rv = [lax.rev(v, (0,)) for v in reversed(bvs)]
    ks = aks + rk
    vs = avs + rv
    while len(ks) // 2 >= cap and len(ks) > 1:
        h = len(ks) // 2
        nk, nv = [], []
        for i in range(h):
            lk, lv, _, _ = _cmp_swap(ks[i], vs[i], ks[i + h], vs[i + h])
            nk.append(lk); nv.append(lv)
        ks, vs = nk, nv
    return _bitonic_merge(ks, vs)


def _select_top(buf_k, buf_i, ptr, out_vregs):
    ks, vs = [], []
    for g in range(CAPV):
        k = buf_k[pl.ds(g * L, L)]
        v = buf_i[pl.ds(g * L, L)]
        pos = lax.iota(jnp.int32, L) + g * L
        k = jnp.where(pos < ptr, k, _INF)
        sk, sv = _vsort(k, v)
        ks.append([sk]); vs.append([sv])
    while len(ks) > 1:
        nk, nv = [], []
        for i in range(0, len(ks), 2):
            a, b = _merge_sorted(ks[i], vs[i], ks[i + 1], vs[i + 1],
                                 cap=max(out_vregs, 1))
            nk.append(a); nv.append(b)
        ks, vs = nk, nv
    return ks[0][:out_vregs], vs[0][:out_vregs]


def _lane_extract_f32(v, lane):
    sel = lax.iota(jnp.int32, L) == lane
    return jnp.max(jnp.where(sel, v, -_INF))


SLACK = 2
FINV = 8
TIE_PASSES = 6
IMAX = np.int32(2**31 - 1)


@functools.cache
def _make_knn_select(BN, N, d):
    T = 16 * d
    TV = T // L
    KEEP = TV + SLACK
    rows_per_w = BN // NW
    steps = N // L
    mesh = plsc.VectorSubcoreMesh(core_axis_name="c", subcore_axis_name="s")

    @functools.partial(
        pl.kernel,
        out_type=jax.ShapeDtypeStruct((BN, L), jnp.int32),
        mesh=mesh,
        compiler_params=pltpu.CompilerParams(needs_layout_passes=False),
        scratch_types=[
            pltpu.VMEM((N,), jnp.float32),
            pltpu.VMEM((N,), jnp.float32),
            pltpu.VMEM((BUF,), jnp.float32),
            pltpu.VMEM((BUF,), jnp.int32),
            pltpu.VMEM((144,), jnp.float32),
            pltpu.VMEM((144,), jnp.int32),
            pltpu.VMEM((L,), jnp.int32),
            pltpu.SemaphoreType.DMA,
            pltpu.SemaphoreType.DMA,
        ],
    )
    def knn(dist_hbm, out_hbm, row_a, row_b, bk, bi, stage_k, stage_i,
            out_v, sem_a, sem_b):
        wid = lax.axis_index("s") * 2 + lax.axis_index("c")
        row0 = wid * rows_per_w

        def process(row, row_v):

            def reselect(ptr, thr):
                ks, vs = _select_top(bk, bi, ptr, KEEP)
                for g in range(KEEP):
                    bk[pl.ds(g * L, L)] = ks[g]
                    bi[pl.ds(g * L, L)] = vs[g]
                new_thr = _lane_extract_f32(ks[(T - 1) // L], (T - 1) % L)
                return np.int32(KEEP * L), new_thr

            def step(s, sc):
                ptr, thr = sc
                base = s * BLK
                vals = [row_v[pl.ds(base + g * L, L)] for g in range(GV)]
                masks = [v <= thr for v in vals]
                anym = masks[0]
                for m in masks[1:]:
                    anym = anym | m

                def append(p, t):
                    off = p + jnp.zeros((L,), jnp.int32)
                    trash = CAP + lax.iota(jnp.int32, L)
                    for g in range(GV):
                        m = masks[g]
                        c = plsc.cumsum(m.astype(jnp.int32))
                        cnt = plsc.all_reduce_population_count(m)
                        idx = lax.iota(jnp.int32, L) + (base + g * L)
                        dest = jnp.where(m, off + c - 1, trash)
                        plsc.store_scatter(bk, [dest], vals[g], mask=m)
                        plsc.store_scatter(bi, [dest], idx, mask=m)
                        off = off + cnt
                    p2 = jnp.max(off)
                    return lax.cond(p2 >= TRIG, reselect,
                                    lambda a, b: (a, b), p2, t)

                return lax.cond(jnp.any(anym), append,
                                lambda a, b: (a, b), ptr, thr)

            ptr, _thr = lax.fori_loop(0, N // BLK, step, (np.int32(0), _INF))

            ks, vs = _select_top(bk, bi, ptr, KEEP)
            stage_k[pl.ds(KEEP * L, L)] = jnp.full((L,), _INF, jnp.float32)
            stage_i[pl.ds(KEEP * L, L)] = jnp.full((L,), IMAX, jnp.int32)
            stage_k[pl.ds(0, L)] = jnp.full((L,), -_INF, jnp.float32)
            stage_i[pl.ds(0, L)] = jnp.full((L,), IMAX, jnp.int32)
            for g in range(KEEP):
                stage_k[pl.ds(g * L + 1, L)] = ks[g]
                stage_i[pl.ds(g * L + 1, L)] = vs[g]
            par0 = lax.iota(jnp.int32, L) % 2
            for p in range(TIE_PASSES):
                par = (par0 + p) % 2 == 0
                new_is = []
                for g in range(KEEP):
                    pk = stage_k[pl.ds(g * L, L)]
                    ck = stage_k[pl.ds(g * L + 1, L)]
                    nk = stage_k[pl.ds(g * L + 2, L)]
                    pi = stage_i[pl.ds(g * L, L)]
                    ci = stage_i[pl.ds(g * L + 1, L)]
                    ni = stage_i[pl.ds(g * L + 2, L)]
                    take_next = par & (ck == nk) & (ci > ni)
                    take_prev = (~par) & (pk == ck) & (pi > ci)
                    new_is.append(jnp.where(
                        take_prev, pi, jnp.where(take_next, ni, ci)))
                for g in range(KEEP):
                    stage_i[pl.ds(g * L + 1, L)] = new_is[g]
            picks = lax.iota(jnp.int32, L) * d + 1
            out_v[...] = plsc.load_gather(stage_i, [picks])
            pltpu.sync_copy(out_v, out_hbm.at[row])

        pltpu.async_copy(dist_hbm.at[row0], row_a, sem_a)

        def do_pair(r2, carry):
            ra = row0 + 2 * r2
            pltpu.make_async_copy(dist_hbm.at[ra], row_a, sem_a).wait()
            pltpu.async_copy(dist_hbm.at[ra + 1], row_b, sem_b)
            process(ra, row_a)
            nxt = jnp.minimum(ra + 2, BN - 1)
            pltpu.make_async_copy(dist_hbm.at[ra + 1], row_b, sem_b).wait()
            pltpu.async_copy(dist_hbm.at[nxt], row_a, sem_a)
            process(ra + 1, row_b)
            return carry

        lax.fori_loop(0, rows_per_w // 2, do_pair, 0)
        pltpu.make_async_copy(dist_hbm.at[row0], row_a, sem_a).wait()

    return knn


def _knn(x, d):
    B, C, N, _ = x.shape
    xt = jnp.transpose(x[:, :, :, 0], (0, 2, 1))
    x2 = jnp.sum(xt * xt, axis=-1, keepdims=True)
    dist = x2 - 2.0 * jnp.einsum('bnc,bmc->bnm', xt, xt) + jnp.transpose(x2, (0, 2, 1))
    nn = _make_knn_select(B * N, N, d)(dist.reshape(B * N, N))
    return nn.reshape(B, N, K)


@functools.cache
def _make_gather(B, C, N):
    PAIRS = B * C
    PPW = -(-PAIRS // NW)
    NK = N * K
    CHUNK = 4096
    mesh = plsc.VectorSubcoreMesh(core_axis_name="c", subcore_axis_name="s")

    @functools.partial(
        pl.kernel,
        out_type=jax.ShapeDtypeStruct((PAIRS, NK), jnp.float32),
        mesh=mesh,
        compiler_params=pltpu.CompilerParams(needs_layout_passes=False),
        scratch_types=[
            pltpu.VMEM((NK,), jnp.int32),
            pltpu.VMEM((N,), jnp.float32),
            pltpu.VMEM((CHUNK,), jnp.float32),
        ],
    )
    def gth(x_hbm, idx_hbm, out_hbm, idx_v, tab_v, stage_v):
        wid = lax.axis_index("s") * 2 + lax.axis_index("c")

        def do_pair(pi, carry):
            pair = wid * PPW + pi

            def work(_):
                b = pair // C
                pltpu.sync_copy(idx_hbm.at[b], idx_v)
                pltpu.sync_copy(x_hbm.at[pair], tab_v)

                def do_chunk(ci, c2):
                    def do_grp(t, c3):
                        base = t * 4 * L
                        for u in range(4):
                            iv = idx_v[pl.ds(ci * CHUNK + base + u * L, L)]
                            stage_v[pl.ds(base + u * L, L)] = (
                                plsc.load_gather(tab_v, [iv]))
                        return c3

                    lax.fori_loop(0, CHUNK // (4 * L), do_grp, 0)
                    pltpu.sync_copy(stage_v,
                                    out_hbm.at[pair, pl.ds(ci * CHUNK, CHUNK)])
                    return c2

                lax.fori_loop(0, NK // CHUNK, do_chunk, 0)
                return 0

            if PAIRS % NW:
                lax.cond(pair < PAIRS, work, lambda _: 0, 0)
            else:
                work(0)
            return carry

        lax.fori_loop(0, PPW, do_pair, 0)

    return gth


def _gather(x, idx):
    B, C, N, _ = x.shape
    out = _make_gather(B, C, N)(x[:, :, :, 0].reshape(B * C, N),
                                idx.reshape(B, N * K))
    return out.reshape(B, C, N, K)


def _conv(x, W, b):
    return jnp.einsum('bcnk,oc->bonk', x, W) + b[None, :, None, None]


def _bn(x):
    m = jnp.mean(x, axis=(0, 2, 3), keepdims=True)
    v = jnp.mean((x - m) ** 2, axis=(0, 2, 3), keepdims=True)
    return (x - m) / jnp.sqrt(v + 1e-5)


def _edge_conv(x, nn_idx, W, b):
    xj = _gather(x, nn_idx)
    xi = jnp.broadcast_to(x, xj.shape)
    h = jnp.concatenate([xi, xj - xi], axis=1)
    h = jax.nn.relu(_bn(_conv(h, W, b)))
    return jnp.max(h, axis=-1, keepdims=True)


def _final_conv_body(x_ref, w_ref, b_ref, o_ref):
    o_ref[...] = jnp.dot(x_ref[...], w_ref[...],
                         preferred_element_type=jnp.float32) + b_ref[...]


def _final_conv(h, W, b):
    B, C, N, _ = h.shape
    O = W.shape[0]
    x = jnp.transpose(h[:, :, :, 0], (0, 2, 1)).reshape(B * N, C)
    out = pl.pallas_call(
        _final_conv_body,
        out_shape=jax.ShapeDtypeStruct((B * N, O), jnp.float32),
        grid=(B * N // 2048,),
        in_specs=[
            pl.BlockSpec((2048, C), lambda i: (i, 0)),
            pl.BlockSpec((C, O), lambda i: (0, 0)),
            pl.BlockSpec((1, O), lambda i: (0, 0)),
        ],
        out_specs=pl.BlockSpec((2048, O), lambda i: (i, 0)),
    )(x, W.T, b.reshape(1, O))
    return out.reshape(B, N, O)


def kernel(inputs, W_head, b_head, W_blk, b_blk, W_fus, b_fus, W_p1, b_p1, W_p2, b_p2, W_p3, b_p3):
    nn_idx = _knn(inputs[:, 0:3], 1)
    x = _edge_conv(inputs, nn_idx, W_head, b_head)
    feats = [x]
    for i in range(N_BLOCKS - 1):
        xin = feats[-1]
        idx = _knn(xin, 1 + i)
        feats.append(_edge_conv(xin, idx, W_blk[i], b_blk[i]) + xin)
    feats = jnp.concatenate(feats, axis=1)
    fusion = jax.nn.relu(_bn(_conv(feats, W_fus, b_fus)))
    fusion = jnp.max(fusion, axis=(2, 3), keepdims=True)
    fusion = jnp.broadcast_to(fusion, (fusion.shape[0], fusion.shape[1], feats.shape[2], 1))
    h = jnp.concatenate([fusion, feats], axis=1)
    h = jax.nn.relu(_bn(_conv(h, W_p1, b_p1)))
    h = jax.nn.relu(_bn(_conv(h, W_p2, b_p2)))
    return _final_conv(h, W_p3, b_p3)


# speedup vs baseline: 1.1709x; 1.0004x over previous
import functools

import numpy as np

import jax
import jax.numpy as jnp
from jax import lax
from jax.experimental import pallas as pl
from jax.experimental.pallas import tpu as pltpu
from jax.experimental.pallas import tpu_sc as plsc

K = 16
N_BLOCKS = 7

L = 16
NW = 32
CAPV = 16
CAP = CAPV * L
GV = 4
BLK = GV * L
TRIG = CAP - BLK
BUF = CAP + L

_INF = np.float32(np.inf)


def _vsort(k, v):
    return plsc.sort_key_val(k, v)


def _cmp_swap(ka, va, kb, vb):
    m = ka <= kb
    return (jnp.where(m, ka, kb), jnp.where(m, va, vb),
            jnp.where(m, kb, ka), jnp.where(m, vb, va))


def _bitonic_merge(ks, vs):
    m = len(ks)
    if m == 1:
        k2, v2 = _vsort(ks[0], vs[0])
        return [k2], [v2]
    h = m // 2
    lo_k, lo_v, hi_k, hi_v = [], [], [], []
    for i in range(h):
        lk, lv, hk, hv = _cmp_swap(ks[i], vs[i], ks[i + h], vs[i + h])
        lo_k.append(lk); lo_v.append(lv); hi_k.append(hk); hi_v.append(hv)
    ak, av = _bitonic_merge(lo_k, lo_v)
    bk, bv = _bitonic_merge(hi_k, hi_v)
    return ak + bk, av + bv


def _merge_sorted(aks, avs, bks, bvs, cap):
    rk = [lax.rev(k, (0,)) for k in reversed(bks)]
    rv = [lax.rev(v, (0,)) for v in reversed(bvs)]
    ks = aks + rk
    vs = avs + rv
    while len(ks) // 2 >= cap and len(ks) > 1:
        h = len(ks) // 2
        nk, nv = [], []
        for i in range(h):
            lk, lv, _, _ = _cmp_swap(ks[i], vs[i], ks[i + h], vs[i + h])
            nk.append(lk); nv.append(lv)
        ks, vs = nk, nv
    return _bitonic_merge(ks, vs)


def _select_top(buf_k, buf_i, ptr, out_vregs):
    ks, vs = [], []
    for g in range(CAPV):
        k = buf_k[pl.ds(g * L, L)]
        v = buf_i[pl.ds(g * L, L)]
        pos = lax.iota(jnp.int32, L) + g * L
        k = jnp.where(pos < ptr, k, _INF)
        sk, sv = _vsort(k, v)
        ks.append([sk]); vs.append([sv])
    while len(ks) > 1:
        nk, nv = [], []
        for i in range(0, len(ks), 2):
            a, b = _merge_sorted(ks[i], vs[i], ks[i + 1], vs[i + 1],
                                 cap=max(out_vregs, 1))
            nk.append(a); nv.append(b)
        ks, vs = nk, nv
    return ks[0][:out_vregs], vs[0][:out_vregs]


def _lane_extract_f32(v, lane):
    sel = lax.iota(jnp.int32, L) == lane
    return jnp.max(jnp.where(sel, v, -_INF))


SLACK = 2
TIE_PASSES = 6
IMAX = np.int32(2**31 - 1)


@functools.cache
def _make_knn_select(BN, N, d):
    T = 16 * d
    TV = T // L
    KEEP = TV + SLACK
    rows_per_w = BN // NW
    mesh = plsc.VectorSubcoreMesh(core_axis_name="c", subcore_axis_name="s")

    @functools.partial(
        pl.kernel,
        out_type=jax.ShapeDtypeStruct((BN, L), jnp.int32),
        mesh=mesh,
        compiler_params=pltpu.CompilerParams(needs_layout_passes=False),
        scratch_types=[
            pltpu.VMEM((N,), jnp.float32),
            pltpu.VMEM((N,), jnp.float32),
            pltpu.VMEM((BUF,), jnp.float32),
            pltpu.VMEM((BUF,), jnp.int32),
            pltpu.VMEM((144,), jnp.float32),
            pltpu.VMEM((144,), jnp.int32),
            pltpu.VMEM((L,), jnp.int32),
            pltpu.SemaphoreType.DMA,
            pltpu.SemaphoreType.DMA,
        ],
    )
    def knn(dist_hbm, out_hbm, row_a, row_b, bk, bi, stage_k, stage_i,
            out_v, sem_a, sem_b):
        wid = lax.axis_index("s") * 2 + lax.axis_index("c")
        row0 = wid * rows_per_w

        def process(row, row_v):

            def reselect(ptr, thr):
                ks, vs = _select_top(bk, bi, ptr, KEEP)
                for g in range(KEEP):
                    bk[pl.ds(g * L, L)] = ks[g]
                    bi[pl.ds(g * L, L)] = vs[g]
                new_thr = _lane_extract_f32(ks[(T - 1) // L], (T - 1) % L)
                return np.int32(KEEP * L), new_thr

            def step(s, sc):
                ptr, thr = sc
                base = s * BLK
                vals = [row_v[pl.ds(base + g * L, L)] for g in range(GV)]
                masks = [v <= thr for v in vals]
                anym = masks[0]
                for m in masks[1:]:
                    anym = anym | m

                def append(p, t):
                    off = p + jnp.zeros((L,), jnp.int32)
                    trash = CAP + lax.iota(jnp.int32, L)
                    for g in range(GV):
                        m = masks[g]
                        c = plsc.cumsum(m.astype(jnp.int32))
                        cnt = plsc.all_reduce_population_count(m)
                        idx = lax.iota(jnp.int32, L) + (base + g * L)
                        dest = jnp.where(m, off + c - 1, trash)
                        plsc.store_scatter(bk, [dest], vals[g], mask=m)
                        plsc.store_scatter(bi, [dest], idx, mask=m)
                        off = off + cnt
                    p2 = jnp.max(off)
                    return lax.cond(p2 >= TRIG, reselect,
                                    lambda a, b: (a, b), p2, t)

                return lax.cond(jnp.any(anym), append,
                                lambda a, b: (a, b), ptr, thr)

            ptr, _thr = lax.fori_loop(0, N // BLK, step, (np.int32(0), _INF))

            ks, vs = _select_top(bk, bi, ptr, KEEP)
            stage_k[pl.ds(KEEP * L, L)] = jnp.full((L,), _INF, jnp.float32)
            stage_i[pl.ds(KEEP * L, L)] = jnp.full((L,), IMAX, jnp.int32)
            stage_k[pl.ds(0, L)] = jnp.full((L,), -_INF, jnp.float32)
            stage_i[pl.ds(0, L)] = jnp.full((L,), IMAX, jnp.int32)
            for g in range(KEEP):
                stage_k[pl.ds(g * L + 1, L)] = ks[g]
                stage_i[pl.ds(g * L + 1, L)] = vs[g]
            par0 = lax.iota(jnp.int32, L) % 2
            for p in range(TIE_PASSES):
                par = (par0 + p) % 2 == 0
                new_is = []
                for g in range(KEEP):
                    pk = stage_k[pl.ds(g * L, L)]
                    ck = stage_k[pl.ds(g * L + 1, L)]
                    nk = stage_k[pl.ds(g * L + 2, L)]
                    pi = stage_i[pl.ds(g * L, L)]
                    ci = stage_i[pl.ds(g * L + 1, L)]
                    ni = stage_i[pl.ds(g * L + 2, L)]
                    take_next = par & (ck == nk) & (ci > ni)
                    take_prev = (~par) & (pk == ck) & (pi > ci)
                    new_is.append(jnp.where(
                        take_prev, pi, jnp.where(take_next, ni, ci)))
                for g in range(KEEP):
                    stage_i[pl.ds(g * L + 1, L)] = new_is[g]
            picks = lax.iota(jnp.int32, L) * d + 1
            out_v[...] = plsc.load_gather(stage_i, [picks])
            pltpu.sync_copy(out_v, out_hbm.at[row])

        pltpu.async_copy(dist_hbm.at[row0], row_a, sem_a)

        def do_pair(r2, carry):
            ra = row0 + 2 * r2
            pltpu.make_async_copy(dist_hbm.at[ra], row_a, sem_a).wait()
            pltpu.async_copy(dist_hbm.at[ra + 1], row_b, sem_b)
            process(ra, row_a)
            nxt = jnp.minimum(ra + 2, BN - 1)
            pltpu.make_async_copy(dist_hbm.at[ra + 1], row_b, sem_b).wait()
            pltpu.async_copy(dist_hbm.at[nxt], row_a, sem_a)
            process(ra + 1, row_b)
            return carry

        lax.fori_loop(0, rows_per_w // 2, do_pair, 0)
        pltpu.make_async_copy(dist_hbm.at[row0], row_a, sem_a).wait()

    return knn


def _knn(x, d):
    B, C, N, _ = x.shape
    xt = jnp.transpose(x[:, :, :, 0], (0, 2, 1))
    x2 = jnp.sum(xt * xt, axis=-1, keepdims=True)
    dist = x2 - 2.0 * jnp.einsum('bnc,bmc->bnm', xt, xt) + jnp.transpose(x2, (0, 2, 1))
    nn = _make_knn_select(B * N, N, d)(dist.reshape(B * N, N))
    return nn.reshape(B, N, K)


@functools.cache
def _make_gather(B, C, N):
    PAIRS = B * C
    PPW = -(-PAIRS // NW)
    NK = N * K
    CHUNK = 4096
    mesh = plsc.VectorSubcoreMesh(core_axis_name="c", subcore_axis_name="s")

    @functools.partial(
        pl.kernel,
        out_type=jax.ShapeDtypeStruct((PAIRS, NK), jnp.float32),
        mesh=mesh,
        compiler_params=pltpu.CompilerParams(needs_layout_passes=False),
        scratch_types=[
            pltpu.VMEM((NK,), jnp.int32),
            pltpu.VMEM((N,), jnp.float32),
            pltpu.VMEM((CHUNK,), jnp.float32),
        ],
    )
    def gth(x_hbm, idx_hbm, out_hbm, idx_v, tab_v, stage_v):
        wid = lax.axis_index("s") * 2 + lax.axis_index("c")

        def do_pair(pi, carry):
            pair = wid * PPW + pi

            def work(_):
                b = pair // C
                pltpu.sync_copy(idx_hbm.at[b], idx_v)
                pltpu.sync_copy(x_hbm.at[pair], tab_v)

                def do_chunk(ci, c2):
                    def do_grp(t, c3):
                        base = t * 4 * L
                        for u in range(4):
                            iv = idx_v[pl.ds(ci * CHUNK + base + u * L, L)]
                            stage_v[pl.ds(base + u * L, L)] = (
                                plsc.load_gather(tab_v, [iv]))
                        return c3

                    lax.fori_loop(0, CHUNK // (4 * L), do_grp, 0)
                    pltpu.sync_copy(stage_v,
                                    out_hbm.at[pair, pl.ds(ci * CHUNK, CHUNK)])
                    return c2

                lax.fori_loop(0, NK // CHUNK, do_chunk, 0)
                return 0

            if PAIRS % NW:
                lax.cond(pair < PAIRS, work, lambda _: 0, 0)
            else:
                work(0)
            return carry

        lax.fori_loop(0, PPW, do_pair, 0)

    return gth


def _gather(x, idx):
    B, C, N, _ = x.shape
    out = _make_gather(B, C, N)(x[:, :, :, 0].reshape(B * C, N),
                                idx.reshape(B, N * K))
    return out.reshape(B, C, N, K)


def _conv(x, W, b):
    return jnp.einsum('bcnk,oc->bonk', x, W) + b[None, :, None, None]


def _bn(x):
    m = jnp.mean(x, axis=(0, 2, 3), keepdims=True)
    v = jnp.mean((x - m) ** 2, axis=(0, 2, 3), keepdims=True)
    return (x - m) / jnp.sqrt(v + 1e-5)


def _edge_conv(x, nn_idx, W, b):
    xj = _gather(x, nn_idx)
    xi = jnp.broadcast_to(x, xj.shape)
    h = jnp.concatenate([xi, xj - xi], axis=1)
    h = jax.nn.relu(_bn(_conv(h, W, b)))
    return jnp.max(h, axis=-1, keepdims=True)


def _final_conv_body(x_ref, w_ref, b_ref, o_ref):
    o_ref[...] = jnp.dot(x_ref[...], w_ref[...],
                         preferred_element_type=jnp.float32) + b_ref[...]


def _final_conv(h, W, b):
    B, C, N, _ = h.shape
    O = W.shape[0]
    x = jnp.transpose(h[:, :, :, 0], (0, 2, 1)).reshape(B * N, C)
    out = pl.pallas_call(
        _final_conv_body,
        out_shape=jax.ShapeDtypeStruct((B * N, O), jnp.float32),
        grid=(B * N // 2048,),
        in_specs=[
            pl.BlockSpec((2048, C), lambda i: (i, 0)),
            pl.BlockSpec((C, O), lambda i: (0, 0)),
            pl.BlockSpec((1, O), lambda i: (0, 0)),
        ],
        out_specs=pl.BlockSpec((2048, O), lambda i: (i, 0)),
    )(x, W.T, b.reshape(1, O))
    return out.reshape(B, N, O)


def kernel(inputs, W_head, b_head, W_blk, b_blk, W_fus, b_fus, W_p1, b_p1, W_p2, b_p2, W_p3, b_p3):
    nn_idx = _knn(inputs[:, 0:3], 1)
    x = _edge_conv(inputs, nn_idx, W_head, b_head)
    feats = [x]
    for i in range(N_BLOCKS - 1):
        xin = feats[-1]
        idx = _knn(xin, 1 + i)
        feats.append(_edge_conv(xin, idx, W_blk[i], b_blk[i]) + xin)
    feats = jnp.concatenate(feats, axis=1)
    fusion = jax.nn.relu(_bn(_conv(feats, W_fus, b_fus)))
    fusion = jnp.max(fusion, axis=(2, 3), keepdims=True)
    fusion = jnp.broadcast_to(fusion, (fusion.shape[0], fusion.shape[1], feats.shape[2], 1))
    h = jnp.concatenate([fusion, feats], axis=1)
    h = jax.nn.relu(_bn(_conv(h, W_p1, b_p1)))
    h = jax.nn.relu(_bn(_conv(h, W_p2, b_p2)))
    return _final_conv(h, W_p3, b_p3)
